# baseline (speedup 1.0000x reference)
import functools

import jax
import jax.numpy as jnp
from jax import lax
from jax.experimental import pallas as pl
from jax.experimental.pallas import tpu as pltpu
from jax.experimental.pallas import tpu_sc as plsc

_SC_FRACTION_NUM = 3
_SC_FRACTION_DEN = 4


def _sc_gather(xf, idx, total, S, V, N, D):
    info = plsc.get_sparse_core_info()
    NC, NS, L = info.num_cores, info.num_subcores, info.num_lanes
    NW = NC * NS
    b_per_w = total // NW
    n_active = S // b_per_w

    mesh = plsc.VectorSubcoreMesh(core_axis_name="c", subcore_axis_name="s")

    @functools.partial(
        pl.kernel,
        mesh=mesh,
        out_type=jax.ShapeDtypeStruct((total, D), jnp.float32),
        scratch_types=[
            pltpu.VMEM((b_per_w,), jnp.int32),
            pltpu.VMEM((b_per_w, D), jnp.float32),
            pltpu.SemaphoreType.DMA,
        ],
    )
    def gather_k(x_hbm, idx_hbm, out_hbm, idx_v, rows_v, sem):
        wid = lax.axis_index("s") * NC + lax.axis_index("c")

        @pl.when(wid < n_active)
        def _active():
            base = wid * b_per_w
            b = base // N
            pltpu.sync_copy(idx_hbm.at[b, pl.ds(base % N, b_per_w)], idx_v)
            for i in range(b_per_w // L):
                idx_v[pl.ds(i * L, L)] = idx_v[pl.ds(i * L, L)] + b * V
            pltpu.async_copy(x_hbm.at[idx_v], rows_v, sem).wait()
            pltpu.sync_copy(rows_v, out_hbm.at[pl.ds(base, b_per_w)])

    return gather_k(xf, idx)


def _tc_fill(x3, idx_tc, partial3, S, M):
    SL, LN = x3.shape[1], x3.shape[2]

    def body(idx_ref, x_ref, partial_ref, out_ref):
        out_ref[...] = x_ref[...]

    grid_spec = pltpu.PrefetchScalarGridSpec(
        num_scalar_prefetch=1,
        grid=(M,),
        in_specs=[
            pl.BlockSpec((1, SL, LN), lambda i, idx: (idx[i], 0, 0)),
            pl.BlockSpec(memory_space=pltpu.MemorySpace.HBM),
        ],
        out_specs=pl.BlockSpec((1, SL, LN), lambda i, idx: (S + i, 0, 0)),
    )
    return pl.pallas_call(
        body,
        grid_spec=grid_spec,
        out_shape=jax.ShapeDtypeStruct(partial3.shape, partial3.dtype),
        input_output_aliases={2: 0},
    )(idx_tc, x3, partial3)


def kernel(x, idx):
    B, V, D = x.shape
    _, N = idx.shape
    total = B * N
    S = total * _SC_FRACTION_NUM // _SC_FRACTION_DEN
    M = total - S

    xf = x.reshape(B * V, D)
    idx32 = idx.astype(jnp.int32)
    partial = _sc_gather(xf, idx32, total, S, V, N, D)

    offs = (jnp.arange(total, dtype=jnp.int32) // N) * V
    idx_tc = (idx32.reshape(total) + offs)[S:]
    x3 = x.reshape(B * V, 8, D // 8)
    out = _tc_fill(x3, idx_tc, partial.reshape(total, 8, D // 8), S, M)
    return out.reshape(B, N, D)


# speedup vs baseline: 13.7270x; 13.7270x over previous
import functools

import jax
import jax.numpy as jnp
from jax import lax
from jax.experimental import pallas as pl
from jax.experimental.pallas import tpu as pltpu
from jax.experimental.pallas import tpu_sc as plsc


def kernel(x, idx):
    B, V, D = x.shape
    _, N = idx.shape
    total = B * N

    info = plsc.get_sparse_core_info()
    NC, NS, L = info.num_cores, info.num_subcores, info.num_lanes
    NW = NC * NS
    b_per_w = total // NW

    mesh = plsc.VectorSubcoreMesh(core_axis_name="c", subcore_axis_name="s")

    @functools.partial(
        pl.kernel,
        mesh=mesh,
        out_type=jax.ShapeDtypeStruct((total, D), jnp.float32),
        scratch_types=[
            pltpu.VMEM((b_per_w,), jnp.int32),
            pltpu.VMEM((b_per_w, D), jnp.float32),
            pltpu.SemaphoreType.DMA,
        ],
    )
    def gather_k(x_hbm, idx_hbm, out_hbm, idx_v, rows_v, sem):
        wid = lax.axis_index("s") * NC + lax.axis_index("c")
        base = wid * b_per_w
        b = base // N
        pltpu.sync_copy(idx_hbm.at[b, pl.ds(base % N, b_per_w)], idx_v)
        for i in range(b_per_w // L):
            idx_v[pl.ds(i * L, L)] = idx_v[pl.ds(i * L, L)] + b * V
        pltpu.async_copy(x_hbm.at[idx_v], rows_v, sem).wait()
        pltpu.sync_copy(rows_v, out_hbm.at[pl.ds(base, b_per_w)])

    xf = x.reshape(B * V, D)
    out = gather_k(xf, idx.astype(jnp.int32))
    return out.reshape(B, N, D)
